# baseline (device time: 25345 ns/iter reference)
import jax
import jax.numpy as jnp
from jax import lax
from jax.experimental import pallas as pl
from jax.experimental.pallas import tpu as pltpu

N_CHUNKS = 8


def kernel(x):
    m, n = x.shape
    nh = n // 2
    hm = m // 2
    cr = hm // N_CHUNKS

    def body(x_ref, out_ref, stage_f32, local_f32, sendx_buf, recvx_buf,
             recvy_buf, in_sems, local_sem,
             x_send_sems, x_recv_sems, y_send_sems, y_recv_sems):
        my_x = lax.axis_index("x")
        my_y = lax.axis_index("y")
        other_x = 1 - my_x
        other_y = 1 - my_y

        barrier_sem = pltpu.get_barrier_semaphore()
        pl.semaphore_signal(
            barrier_sem, inc=1,
            device_id=(other_x, my_y), device_id_type=pl.DeviceIdType.MESH,
        )
        pl.semaphore_signal(
            barrier_sem, inc=1,
            device_id=(my_x, other_y), device_id_type=pl.DeviceIdType.MESH,
        )
        pl.semaphore_wait(barrier_sem, 2)

        def in_copy(c):
            return pltpu.make_async_copy(
                x_ref.at[pl.ds(my_y * hm + c * cr, cr), pl.ds(other_x * nh, nh)],
                stage_f32.at[pl.ds(c * cr, cr)],
                in_sems.at[c],
            )

        local_copy = pltpu.make_async_copy(
            x_ref.at[:, pl.ds(my_x * nh, nh)], local_f32, local_sem
        )

        def x_rdma(c):
            return pltpu.make_async_remote_copy(
                src_ref=sendx_buf.at[pl.ds(c * cr, cr)],
                dst_ref=recvx_buf.at[pl.ds(c * cr, cr)],
                send_sem=x_send_sems.at[c],
                recv_sem=x_recv_sems.at[c],
                device_id=(other_x, my_y),
                device_id_type=pl.DeviceIdType.MESH,
            )

        def y_rdma(c):
            return pltpu.make_async_remote_copy(
                src_ref=recvx_buf.at[pl.ds(c * cr, cr)],
                dst_ref=recvy_buf.at[pl.ds(c * cr, cr)],
                send_sem=y_send_sems.at[c],
                recv_sem=y_recv_sems.at[c],
                device_id=(my_x, other_y),
                device_id_type=pl.DeviceIdType.MESH,
            )

        for c in range(N_CHUNKS):
            in_copy(c).start()
        local_copy.start()
        for c in range(N_CHUNKS):
            in_copy(c).wait()
            sendx_buf[pl.ds(c * cr, cr), :] = stage_f32[
                pl.ds(c * cr, cr), :
            ].astype(jnp.bfloat16)
            x_rdma(c).start()

        local_copy.wait()
        out_ref[pl.ds(my_x * m, m), :] = local_f32[...].astype(jnp.bfloat16)

        base_x = other_x * m + my_y * hm
        for c in range(N_CHUNKS):
            x_rdma(c).wait_recv()
            y_rdma(c).start()
            out_ref[pl.ds(base_x + c * cr, cr), :] = recvx_buf[
                pl.ds(c * cr, cr), :
            ]

        base_y = other_x * m + other_y * hm
        for c in range(N_CHUNKS):
            y_rdma(c).wait_recv()
            out_ref[pl.ds(base_y + c * cr, cr), :] = recvy_buf[
                pl.ds(c * cr, cr), :
            ]

        for c in range(N_CHUNKS):
            x_rdma(c).wait_send()
            y_rdma(c).wait_send()

    return pl.pallas_call(
        body,
        out_shape=jax.ShapeDtypeStruct((2 * m, nh), jnp.bfloat16),
        in_specs=[pl.BlockSpec(memory_space=pl.ANY)],
        out_specs=pl.BlockSpec(memory_space=pltpu.VMEM),
        scratch_shapes=[
            pltpu.VMEM((hm, nh), jnp.float32),
            pltpu.VMEM((m, nh), jnp.float32),
            pltpu.VMEM((hm, nh), jnp.bfloat16),
            pltpu.VMEM((hm, nh), jnp.bfloat16),
            pltpu.VMEM((hm, nh), jnp.bfloat16),
            pltpu.SemaphoreType.DMA((N_CHUNKS,)),
            pltpu.SemaphoreType.DMA,
            pltpu.SemaphoreType.DMA((N_CHUNKS,)),
            pltpu.SemaphoreType.DMA((N_CHUNKS,)),
            pltpu.SemaphoreType.DMA((N_CHUNKS,)),
            pltpu.SemaphoreType.DMA((N_CHUNKS,)),
        ],
        compiler_params=pltpu.CompilerParams(collective_id=0),
    )(x)


# device time: 25250 ns/iter; 1.0038x vs baseline; 1.0038x over previous
import jax
import jax.numpy as jnp
from jax import lax
from jax.experimental import pallas as pl
from jax.experimental.pallas import tpu as pltpu

N_CHUNKS = 16


def kernel(x):
    m, n = x.shape
    nh = n // 2
    hm = m // 2
    cr = hm // N_CHUNKS

    def body(x_ref, out_ref, stage_f32, localo_f32, local_bf,
             sendx_buf, recvx_buf, recvy_buf,
             in_sems, localo_sem, outl_sem, outx_sems, outy_sems,
             x_send_sems, x_recv_sems, y_send_sems, y_recv_sems):
        my_x = lax.axis_index("x")
        my_y = lax.axis_index("y")
        other_x = 1 - my_x
        other_y = 1 - my_y

        barrier_sem = pltpu.get_barrier_semaphore()
        pl.semaphore_signal(
            barrier_sem, inc=1,
            device_id=(other_x, my_y), device_id_type=pl.DeviceIdType.MESH,
        )
        pl.semaphore_signal(
            barrier_sem, inc=1,
            device_id=(my_x, other_y), device_id_type=pl.DeviceIdType.MESH,
        )
        pl.semaphore_wait(barrier_sem, 2)

        def in_copy(c):
            return pltpu.make_async_copy(
                x_ref.at[pl.ds(my_y * hm + c * cr, cr), :],
                stage_f32.at[pl.ds(c * cr, cr)],
                in_sems.at[c],
            )

        localo_copy = pltpu.make_async_copy(
            x_ref.at[pl.ds(other_y * hm, hm), pl.ds(my_x * nh, nh)],
            localo_f32,
            localo_sem,
        )

        def x_rdma(c):
            return pltpu.make_async_remote_copy(
                src_ref=sendx_buf.at[pl.ds(c * cr, cr)],
                dst_ref=recvx_buf.at[pl.ds(c * cr, cr)],
                send_sem=x_send_sems.at[c],
                recv_sem=x_recv_sems.at[c],
                device_id=(other_x, my_y),
                device_id_type=pl.DeviceIdType.MESH,
            )

        def y_rdma(c):
            return pltpu.make_async_remote_copy(
                src_ref=recvx_buf.at[pl.ds(c * cr, cr)],
                dst_ref=recvy_buf.at[pl.ds(c * cr, cr)],
                send_sem=y_send_sems.at[c],
                recv_sem=y_recv_sems.at[c],
                device_id=(my_x, other_y),
                device_id_type=pl.DeviceIdType.MESH,
            )

        for c in range(N_CHUNKS):
            in_copy(c).start()
        localo_copy.start()

        for c in range(N_CHUNKS):
            in_copy(c).wait()
            sendx_buf[pl.ds(c * cr, cr), :] = stage_f32[
                pl.ds(c * cr, cr), pl.ds(other_x * nh, nh)
            ].astype(jnp.bfloat16)
            x_rdma(c).start()
            local_bf[pl.ds(my_y * hm + c * cr, cr), :] = stage_f32[
                pl.ds(c * cr, cr), pl.ds(my_x * nh, nh)
            ].astype(jnp.bfloat16)

        localo_copy.wait()
        local_bf[pl.ds(other_y * hm, hm), :] = localo_f32[...].astype(
            jnp.bfloat16
        )
        out_local = pltpu.make_async_copy(
            local_bf, out_ref.at[pl.ds(my_x * m, m), :], outl_sem
        )
        out_local.start()

        base_x = other_x * m + my_y * hm
        for c in range(N_CHUNKS):
            x_rdma(c).wait_recv()
            y_rdma(c).start()
            pltpu.make_async_copy(
                recvx_buf.at[pl.ds(c * cr, cr)],
                out_ref.at[pl.ds(base_x + c * cr, cr), :],
                outx_sems.at[c],
            ).start()

        base_y = other_x * m + other_y * hm
        for c in range(N_CHUNKS):
            y_rdma(c).wait_recv()
            pltpu.make_async_copy(
                recvy_buf.at[pl.ds(c * cr, cr)],
                out_ref.at[pl.ds(base_y + c * cr, cr), :],
                outy_sems.at[c],
            ).start()

        out_local.wait()
        for c in range(N_CHUNKS):
            pltpu.make_async_copy(
                recvx_buf.at[pl.ds(c * cr, cr)],
                out_ref.at[pl.ds(base_x + c * cr, cr), :],
                outx_sems.at[c],
            ).wait()
            pltpu.make_async_copy(
                recvy_buf.at[pl.ds(c * cr, cr)],
                out_ref.at[pl.ds(base_y + c * cr, cr), :],
                outy_sems.at[c],
            ).wait()
            x_rdma(c).wait_send()
            y_rdma(c).wait_send()

    return pl.pallas_call(
        body,
        out_shape=jax.ShapeDtypeStruct((2 * m, nh), jnp.bfloat16),
        in_specs=[pl.BlockSpec(memory_space=pl.ANY)],
        out_specs=pl.BlockSpec(memory_space=pl.ANY),
        scratch_shapes=[
            pltpu.VMEM((hm, n), jnp.float32),
            pltpu.VMEM((hm, nh), jnp.float32),
            pltpu.VMEM((m, nh), jnp.bfloat16),
            pltpu.VMEM((hm, nh), jnp.bfloat16),
            pltpu.VMEM((hm, nh), jnp.bfloat16),
            pltpu.VMEM((hm, nh), jnp.bfloat16),
            pltpu.SemaphoreType.DMA((N_CHUNKS,)),
            pltpu.SemaphoreType.DMA,
            pltpu.SemaphoreType.DMA,
            pltpu.SemaphoreType.DMA((N_CHUNKS,)),
            pltpu.SemaphoreType.DMA((N_CHUNKS,)),
            pltpu.SemaphoreType.DMA((N_CHUNKS,)),
            pltpu.SemaphoreType.DMA((N_CHUNKS,)),
            pltpu.SemaphoreType.DMA((N_CHUNKS,)),
            pltpu.SemaphoreType.DMA((N_CHUNKS,)),
        ],
        compiler_params=pltpu.CompilerParams(collective_id=0),
    )(x)


# device time: 25227 ns/iter; 1.0047x vs baseline; 1.0009x over previous
import jax
import jax.numpy as jnp
from jax import lax
from jax.experimental import pallas as pl
from jax.experimental.pallas import tpu as pltpu

N_CHUNKS = 16


def kernel(x):
    m, n = x.shape
    nh = n // 2
    hm = m // 2
    cr = hm // N_CHUNKS

    def body(x_ref, out_ref, stage_f32, localo_f32, local_bf,
             sendx_buf, recvx_buf, recvy_buf,
             in_sems, localo_sem, outl_sem, outx_sems, outy_sems,
             x_send_sems, x_recv_sems, y_send_sems, y_recv_sems):
        my_x = lax.axis_index("x")
        my_y = lax.axis_index("y")
        other_x = 1 - my_x
        other_y = 1 - my_y

        def in_copy(c):
            return pltpu.make_async_copy(
                x_ref.at[pl.ds(my_y * hm + c * cr, cr), :],
                stage_f32.at[pl.ds(c * cr, cr)],
                in_sems.at[c],
            )

        localo_copy = pltpu.make_async_copy(
            x_ref.at[pl.ds(other_y * hm, hm), pl.ds(my_x * nh, nh)],
            localo_f32,
            localo_sem,
        )

        def x_rdma(c):
            return pltpu.make_async_remote_copy(
                src_ref=sendx_buf.at[pl.ds(c * cr, cr)],
                dst_ref=recvx_buf.at[pl.ds(c * cr, cr)],
                send_sem=x_send_sems.at[c],
                recv_sem=x_recv_sems.at[c],
                device_id=(other_x, my_y),
                device_id_type=pl.DeviceIdType.MESH,
            )

        def y_rdma(c):
            return pltpu.make_async_remote_copy(
                src_ref=recvx_buf.at[pl.ds(c * cr, cr)],
                dst_ref=recvy_buf.at[pl.ds(c * cr, cr)],
                send_sem=y_send_sems.at[c],
                recv_sem=y_recv_sems.at[c],
                device_id=(my_x, other_y),
                device_id_type=pl.DeviceIdType.MESH,
            )

        for c in range(N_CHUNKS):
            in_copy(c).start()
        localo_copy.start()

        barrier_sem = pltpu.get_barrier_semaphore()
        pl.semaphore_signal(
            barrier_sem, inc=1,
            device_id=(other_x, my_y), device_id_type=pl.DeviceIdType.MESH,
        )
        pl.semaphore_signal(
            barrier_sem, inc=1,
            device_id=(my_x, other_y), device_id_type=pl.DeviceIdType.MESH,
        )
        pl.semaphore_wait(barrier_sem, 2)

        for c in range(N_CHUNKS):
            in_copy(c).wait()
            sendx_buf[pl.ds(c * cr, cr), :] = stage_f32[
                pl.ds(c * cr, cr), pl.ds(other_x * nh, nh)
            ].astype(jnp.bfloat16)
            x_rdma(c).start()

        local_bf[pl.ds(my_y * hm, hm), :] = stage_f32[
            :, pl.ds(my_x * nh, nh)
        ].astype(jnp.bfloat16)
        localo_copy.wait()
        local_bf[pl.ds(other_y * hm, hm), :] = localo_f32[...].astype(
            jnp.bfloat16
        )
        out_local = pltpu.make_async_copy(
            local_bf, out_ref.at[pl.ds(my_x * m, m), :], outl_sem
        )
        out_local.start()

        base_x = other_x * m + my_y * hm
        for c in range(N_CHUNKS):
            x_rdma(c).wait_recv()
            y_rdma(c).start()
            pltpu.make_async_copy(
                recvx_buf.at[pl.ds(c * cr, cr)],
                out_ref.at[pl.ds(base_x + c * cr, cr), :],
                outx_sems.at[c],
            ).start()

        base_y = other_x * m + other_y * hm
        for c in range(N_CHUNKS):
            y_rdma(c).wait_recv()
            pltpu.make_async_copy(
                recvy_buf.at[pl.ds(c * cr, cr)],
                out_ref.at[pl.ds(base_y + c * cr, cr), :],
                outy_sems.at[c],
            ).start()

        out_local.wait()
        for c in range(N_CHUNKS):
            pltpu.make_async_copy(
                recvx_buf.at[pl.ds(c * cr, cr)],
                out_ref.at[pl.ds(base_x + c * cr, cr), :],
                outx_sems.at[c],
            ).wait()
            pltpu.make_async_copy(
                recvy_buf.at[pl.ds(c * cr, cr)],
                out_ref.at[pl.ds(base_y + c * cr, cr), :],
                outy_sems.at[c],
            ).wait()
            x_rdma(c).wait_send()
            y_rdma(c).wait_send()

    return pl.pallas_call(
        body,
        out_shape=jax.ShapeDtypeStruct((2 * m, nh), jnp.bfloat16),
        in_specs=[pl.BlockSpec(memory_space=pl.ANY)],
        out_specs=pl.BlockSpec(memory_space=pl.ANY),
        scratch_shapes=[
            pltpu.VMEM((hm, n), jnp.float32),
            pltpu.VMEM((hm, nh), jnp.float32),
            pltpu.VMEM((m, nh), jnp.bfloat16),
            pltpu.VMEM((hm, nh), jnp.bfloat16),
            pltpu.VMEM((hm, nh), jnp.bfloat16),
            pltpu.VMEM((hm, nh), jnp.bfloat16),
            pltpu.SemaphoreType.DMA((N_CHUNKS,)),
            pltpu.SemaphoreType.DMA,
            pltpu.SemaphoreType.DMA,
            pltpu.SemaphoreType.DMA((N_CHUNKS,)),
            pltpu.SemaphoreType.DMA((N_CHUNKS,)),
            pltpu.SemaphoreType.DMA((N_CHUNKS,)),
            pltpu.SemaphoreType.DMA((N_CHUNKS,)),
            pltpu.SemaphoreType.DMA((N_CHUNKS,)),
            pltpu.SemaphoreType.DMA((N_CHUNKS,)),
        ],
        compiler_params=pltpu.CompilerParams(collective_id=0),
    )(x)


# device time: 21828 ns/iter; 1.1611x vs baseline; 1.1557x over previous
import jax
import jax.numpy as jnp
from jax import lax
from jax.experimental import pallas as pl
from jax.experimental.pallas import tpu as pltpu

CHUNK_ROWS = [64] * 16
CHUNK_OFFS = [sum(CHUNK_ROWS[:i]) for i in range(len(CHUNK_ROWS))]
N_CHUNKS = len(CHUNK_ROWS)
assert sum(CHUNK_ROWS) == 1024


def kernel(x):
    m, n = x.shape
    nh = n // 2
    hm = m // 2
    chunks = list(zip(CHUNK_OFFS, CHUNK_ROWS))

    def body(x_ref, out_ref, stage_f32, localo_f32, local_bf,
             sendx_buf, recvx_buf, recvy_buf,
             in_sems, localo_sem, outl_sem, outx_sems, outy_sems,
             x_send_sems, x_recv_sems, y_send_sems, y_recv_sems):
        my_x = lax.axis_index("x")
        my_y = lax.axis_index("y")
        other_x = 1 - my_x
        other_y = 1 - my_y

        def in_copy(c):
            off, sz = chunks[c]
            return pltpu.make_async_copy(
                x_ref.at[pl.ds(my_y * hm + off, sz), :],
                stage_f32.at[pl.ds(off, sz)],
                in_sems.at[c],
            )

        localo_copy = pltpu.make_async_copy(
            x_ref.at[pl.ds(other_y * hm, hm), pl.ds(my_x * nh, nh)],
            localo_f32,
            localo_sem,
        )

        def x_rdma(c):
            off, sz = chunks[c]
            return pltpu.make_async_remote_copy(
                src_ref=sendx_buf.at[pl.ds(off, sz)],
                dst_ref=recvx_buf.at[pl.ds(off, sz)],
                send_sem=x_send_sems.at[c],
                recv_sem=x_recv_sems.at[c],
                device_id=(other_x, my_y),
                device_id_type=pl.DeviceIdType.MESH,
            )

        def y_rdma(c):
            off, sz = chunks[c]
            return pltpu.make_async_remote_copy(
                src_ref=recvx_buf.at[pl.ds(off, sz)],
                dst_ref=recvy_buf.at[pl.ds(off, sz)],
                send_sem=y_send_sems.at[c],
                recv_sem=y_recv_sems.at[c],
                device_id=(my_x, other_y),
                device_id_type=pl.DeviceIdType.MESH,
            )

        def out_x_copy(c):
            off, sz = chunks[c]
            return pltpu.make_async_copy(
                recvx_buf.at[pl.ds(off, sz)],
                out_ref.at[pl.ds(other_x * m + my_y * hm + off, sz), :],
                outx_sems.at[c],
            )

        def out_y_copy(c):
            off, sz = chunks[c]
            return pltpu.make_async_copy(
                recvy_buf.at[pl.ds(off, sz)],
                out_ref.at[pl.ds(other_x * m + other_y * hm + off, sz), :],
                outy_sems.at[c],
            )

        for c in range(N_CHUNKS):
            in_copy(c).start()
        localo_copy.start()

        barrier_sem = pltpu.get_barrier_semaphore()
        pl.semaphore_signal(
            barrier_sem, inc=1,
            device_id=(other_x, my_y), device_id_type=pl.DeviceIdType.MESH,
        )
        pl.semaphore_signal(
            barrier_sem, inc=1,
            device_id=(my_x, other_y), device_id_type=pl.DeviceIdType.MESH,
        )
        pl.semaphore_wait(barrier_sem, 2)

        for c in range(N_CHUNKS):
            off, sz = chunks[c]
            in_copy(c).wait()
            sendx_buf[pl.ds(off, sz), :] = stage_f32[
                pl.ds(off, sz), pl.ds(other_x * nh, nh)
            ].astype(jnp.bfloat16)
            x_rdma(c).start()

        local_bf[pl.ds(my_y * hm, hm), :] = stage_f32[
            :, pl.ds(my_x * nh, nh)
        ].astype(jnp.bfloat16)
        localo_copy.wait()
        local_bf[pl.ds(other_y * hm, hm), :] = localo_f32[...].astype(
            jnp.bfloat16
        )
        out_local = pltpu.make_async_copy(
            local_bf, out_ref.at[pl.ds(my_x * m, m), :], outl_sem
        )
        out_local.start()

        for c in range(N_CHUNKS):
            x_rdma(c).wait_recv()
            y_rdma(c).start()
            out_x_copy(c).start()

        for c in range(N_CHUNKS):
            y_rdma(c).wait_recv()
            out_y_copy(c).start()

        out_local.wait()
        for c in range(N_CHUNKS):
            out_x_copy(c).wait()
            out_y_copy(c).wait()
            x_rdma(c).wait_send()
            y_rdma(c).wait_send()

    x = pltpu.with_memory_space_constraint(x, pltpu.MemorySpace.HBM)
    return pl.pallas_call(
        body,
        out_shape=jax.ShapeDtypeStruct((2 * m, nh), jnp.bfloat16),
        in_specs=[pl.BlockSpec(memory_space=pltpu.MemorySpace.HBM)],
        out_specs=pl.BlockSpec(memory_space=pltpu.MemorySpace.HBM),
        scratch_shapes=[
            pltpu.VMEM((hm, n), jnp.float32),
            pltpu.VMEM((hm, nh), jnp.float32),
            pltpu.VMEM((m, nh), jnp.bfloat16),
            pltpu.VMEM((hm, nh), jnp.bfloat16),
            pltpu.VMEM((hm, nh), jnp.bfloat16),
            pltpu.VMEM((hm, nh), jnp.bfloat16),
            pltpu.SemaphoreType.DMA((N_CHUNKS,)),
            pltpu.SemaphoreType.DMA,
            pltpu.SemaphoreType.DMA,
            pltpu.SemaphoreType.DMA((N_CHUNKS,)),
            pltpu.SemaphoreType.DMA((N_CHUNKS,)),
            pltpu.SemaphoreType.DMA((N_CHUNKS,)),
            pltpu.SemaphoreType.DMA((N_CHUNKS,)),
            pltpu.SemaphoreType.DMA((N_CHUNKS,)),
            pltpu.SemaphoreType.DMA((N_CHUNKS,)),
        ],
        compiler_params=pltpu.CompilerParams(collective_id=0),
    )(x)
